# baseline (speedup 1.0000x reference)
import functools

import jax
import jax.numpy as jnp
from jax import lax
from jax.experimental import pallas as pl
from jax.experimental.pallas import tpu as pltpu, tpu_sc as plsc

_NW = 32
_L = 16
_CHUNK = 128


def _embedding_sc(B, b_per_w, D):
    n_chunks = b_per_w // _CHUNK
    mesh = plsc.VectorSubcoreMesh(core_axis_name="c", subcore_axis_name="s")

    @functools.partial(
        pl.kernel,
        mesh=mesh,
        out_type=jax.ShapeDtypeStruct((B, D), jnp.float32),
        scratch_types=[
            pltpu.VMEM((n_chunks, _CHUNK), jnp.int32),
            pltpu.VMEM((b_per_w,), jnp.int32),
            pltpu.SMEM((b_per_w,), jnp.int32),
            pltpu.VMEM((_CHUNK, 128), jnp.float32),
            pltpu.VMEM((_CHUNK, 128), jnp.float32),
            pltpu.VMEM((_CHUNK, D), jnp.float32),
            pltpu.VMEM((_CHUNK, D), jnp.float32),
            pltpu.SemaphoreType.DMA,
            pltpu.SemaphoreType.DMA,
        ],
    )
    def k(vidx_hbm, off_hbm, table_hbm, out_hbm, vidx_v, off_v, off_s,
          gat0, gat1, rows0, rows1, gsem, osem):
        nc = lax.axis_size("c")
        wid = lax.axis_index("s") * nc + lax.axis_index("c")
        base = wid * b_per_w
        pltpu.sync_copy(vidx_hbm.at[wid], vidx_v)
        pltpu.sync_copy(off_hbm.at[pl.ds(base, b_per_w)], off_v)

        gbufs = [gat0, gat1]
        rbufs = [rows0, rows1]

        def chunk_gather(j, slot):
            return pltpu.make_async_copy(
                table_hbm.at[vidx_v.at[j]], gbufs[slot], gsem
            )

        def unpack_body(c, _):
            vec = off_v[pl.ds(c * _L, _L)]
            for u in range(_L):
                off_s[c * _L + u] = vec[u]
            return ()

        chunk_gather(0, 0).start()
        lax.fori_loop(0, b_per_w // _L, unpack_body, ())

        for j in range(n_chunks):
            if j + 1 < n_chunks:
                chunk_gather(j + 1, (j + 1) % 2).start()
            chunk_gather(j, j % 2).wait()
            if j > 1:
                pltpu.make_async_copy(
                    rbufs[j % 2],
                    out_hbm.at[pl.ds(base + (j - 2) * _CHUNK, _CHUNK)],
                    osem,
                ).wait()
            src = gbufs[j % 2]
            rows = rbufs[j % 2]

            def extract_body(r, _):
                off = off_s[j * _CHUNK + r]
                srow = src.at[r]
                drow = rows.at[r]
                drow[pl.ds(0, _L)] = srow[pl.ds(off, _L)]
                drow[pl.ds(_L, _L)] = srow[pl.ds(off + _L, _L)]
                return ()

            lax.fori_loop(0, _CHUNK, extract_body, ())
            pltpu.async_copy(
                rows, out_hbm.at[pl.ds(base + j * _CHUNK, _CHUNK)], osem
            )
        for j in (n_chunks - 2, n_chunks - 1):
            pltpu.make_async_copy(
                rbufs[j % 2], out_hbm.at[pl.ds(base + j * _CHUNK, _CHUNK)], osem
            ).wait()

    return k


def kernel(input_ids, table):
    B = input_ids.shape[0]
    D = table.shape[1]
    pack = 128 // D
    b_per_w = B // _NW
    ids = input_ids.astype(jnp.int32)
    vidx = (ids // pack).reshape(_NW, b_per_w // _CHUNK, _CHUNK)
    offs = (ids % pack) * D
    tbl4 = table.reshape(table.shape[0] // pack, 128)
    out = _embedding_sc(B, b_per_w, D)(vidx, offs, tbl4)
    return out.reshape(B, 1, D)
